# baseline (speedup 1.0000x reference)
import functools

import jax
import jax.numpy as jnp
from jax import lax
from jax.experimental import pallas as pl
from jax.experimental.pallas import tpu as pltpu
from jax.experimental.pallas import tpu_sc as plsc

N_NODES = 10000
N_HEDGES = 10000
N_INC = 320000
NC, NS = 2, 16
CH = 128
NI_PAD = 327680
NCH = NI_PAD // NS // CH
NB = 5
NB2 = 10
R_ACC = 10240
RPT = R_ACC // NS
ZR = 128

_f32 = jnp.float32
_i32 = jnp.int32


def _sc_pass_body(split, D, *refs):
    (tab, gi, si, out,
     gidx_v, sidx_v, rows_v, zrow_v, acc_sh,
     sems) = refs
    gsem, ssem = sems
    nch = NCHW if split else NCH
    nb = NB2 if split else NB

    c = lax.axis_index("c")
    s = lax.axis_index("s")
    row0 = s * RPT

    z16 = jnp.zeros((16,), _f32)

    def zb(r, _):
        for cc in range(D // 16):
            zrow_v[r, pl.ds(cc * 16, 16)] = z16
        return 0

    lax.fori_loop(0, ZR, zb, 0)
    for t in range(RPT // ZR):
        pltpu.sync_copy(zrow_v, acc_sh.at[pl.ds(row0 + t * ZR, ZR)])

    if split:
        wid = s * NC + c
        pltpu.sync_copy(gi.at[wid], gidx_v)
        pltpu.sync_copy(si.at[wid], sidx_v)
    else:
        pltpu.sync_copy(gi.at[c, s], gidx_v)
        pltpu.sync_copy(si.at[s], sidx_v)

    plsc.subcore_barrier()

    for b in range(nb):
        pltpu.async_copy(tab.at[gidx_v.at[b]], rows_v.at[b], gsem)

    def super_step(it, _):
        j0 = it * nb
        for b in range(nb):
            j = j0 + b
            pltpu.make_async_copy(tab.at[gidx_v.at[j]], rows_v.at[b],
                                  gsem).wait()
            pltpu.async_copy(rows_v.at[b], acc_sh.at[sidx_v.at[j]], ssem,
                             add=True)
        for b in range(nb):
            j = j0 + b
            pltpu.make_async_copy(rows_v.at[b], acc_sh.at[sidx_v.at[j]],
                                  ssem).wait()
        for b in range(nb):
            jn = j0 + nb + b

            @pl.when(jn < nch)
            def _():
                pltpu.async_copy(tab.at[gidx_v.at[jn]], rows_v.at[b], gsem)
        return 0

    lax.fori_loop(0, nch // nb, super_step, 0)

    plsc.subcore_barrier()

    pltpu.sync_copy(acc_sh.at[pl.ds(row0, RPT)],
                    out.at[c, pl.ds(row0, RPT)])


@functools.lru_cache(maxsize=None)
def _sc_pass_fn(split, D):
    mesh = plsc.VectorSubcoreMesh(core_axis_name="c", subcore_axis_name="s",
                                  num_cores=NC, num_subcores=NS)
    nch = NCHW if split else NCH
    nb = NB2 if split else NB
    out_type = jax.ShapeDtypeStruct((NC, R_ACC, D), _f32)
    scratch = [
        pltpu.VMEM((nch, CH), _i32),
        pltpu.VMEM((nch, CH), _i32),
        pltpu.VMEM((nb, CH, D), _f32),
        pltpu.VMEM((ZR, D), _f32),
        pltpu.VMEM_SHARED((R_ACC, D), _f32),
        (pltpu.SemaphoreType.DMA, pltpu.SemaphoreType.DMA),
    ]
    return pl.kernel(functools.partial(_sc_pass_body, split, D),
                     out_type=out_type, mesh=mesh,
                     compiler_params=pltpu.CompilerParams(
                         use_tc_tiling_on_sc=False),
                     scratch_types=tuple(scratch))


def _sc_pass(table, gidx, sidx, split=False):
    return _sc_pass_fn(split, table.shape[1])(table, gidx, sidx)


NCHW = NI_PAD // (NC * NS) // CH


def _counts_body(*refs):
    si, cnt_out, sidx_v, zcnt_v, ones_v, cnt_sh, sems = refs
    gsem, ssem = sems
    c = lax.axis_index("c")
    s = lax.axis_index("s")
    wid = s * NC + c
    row0 = s * RPT

    z16 = jnp.zeros((16,), _f32)
    o16 = jnp.full((16,), 1.0, _f32)

    def zb(r, _):
        zcnt_v[r, pl.ds(0, 16)] = z16
        ones_v[r, pl.ds(0, 16)] = o16
        return 0

    lax.fori_loop(0, ZR, zb, 0)
    for t in range(RPT // ZR):
        pltpu.sync_copy(zcnt_v, cnt_sh.at[pl.ds(row0 + t * ZR, ZR)])
    pltpu.sync_copy(si.at[wid], sidx_v)
    plsc.subcore_barrier()

    def step(j, _):
        pltpu.sync_copy(ones_v, cnt_sh.at[sidx_v.at[j]], add=True)
        return 0

    lax.fori_loop(0, NCHW, step, 0)
    plsc.subcore_barrier()
    pltpu.sync_copy(cnt_sh.at[pl.ds(row0, RPT)],
                    cnt_out.at[c, pl.ds(row0, RPT)])


@functools.lru_cache(maxsize=None)
def _counts_fn():
    mesh = plsc.VectorSubcoreMesh(core_axis_name="c", subcore_axis_name="s",
                                  num_cores=NC, num_subcores=NS)
    scratch = [
        pltpu.VMEM((NCHW, CH), _i32),
        pltpu.VMEM((ZR, 16), _f32),
        pltpu.VMEM((CH, 16), _f32),
        pltpu.VMEM_SHARED((R_ACC, 16), _f32),
        (pltpu.SemaphoreType.DMA, pltpu.SemaphoreType.DMA),
    ]
    return pl.kernel(_counts_body,
                     out_type=jax.ShapeDtypeStruct((NC, R_ACC, 16), _f32),
                     mesh=mesh,
                     compiler_params=pltpu.CompilerParams(
                         use_tc_tiling_on_sc=False),
                     scratch_types=tuple(scratch))


PH = 320


def _ring_loop(tab, gidx_v, sidx_v, rows_v, acc_sh, gsem, ssem, nch):
    for b in range(nb):
        pltpu.async_copy(tab.at[gidx_v.at[b]], rows_v.at[b], gsem)

    def super_step(it, _):
        j0 = it * nb
        for b in range(nb):
            j = j0 + b
            pltpu.make_async_copy(tab.at[gidx_v.at[j]], rows_v.at[b],
                                  gsem).wait()
            pltpu.async_copy(rows_v.at[b], acc_sh.at[sidx_v.at[j]], ssem,
                             add=True)
        for b in range(nb):
            j = j0 + b
            pltpu.make_async_copy(rows_v.at[b], acc_sh.at[sidx_v.at[j]],
                                  ssem).wait()
        for b in range(nb):
            jn = j0 + nb + b

            @pl.when(jn < nch)
            def _():
                pltpu.async_copy(tab.at[gidx_v.at[jn]], rows_v.at[b], gsem)
        return 0

    lax.fori_loop(0, nch // nb, super_step, 0)


def _l1_body(*refs):
    (tab, cnt, gi1, si1, gi2, si2, y_out, va_out,
     gidx_v, sidx_v, rows_v, zrow_v, ybuf_v, c0_v, c1_v, acc_sh,
     sems) = refs
    gsem, ssem = sems
    c = lax.axis_index("c")
    s = lax.axis_index("s")
    row0 = s * RPT
    D = 64

    z16 = jnp.zeros((16,), _f32)

    def zb(r, _):
        for cc in range(D // 16):
            zrow_v[r, pl.ds(cc * 16, 16)] = z16
        return 0

    lax.fori_loop(0, ZR, zb, 0)
    for t in range(RPT // ZR):
        pltpu.sync_copy(zrow_v, acc_sh.at[pl.ds(row0 + t * ZR, ZR)])
    pltpu.sync_copy(gi1.at[c, s], gidx_v)
    pltpu.sync_copy(si1.at[s], sidx_v)
    plsc.subcore_barrier()

    _ring_loop(tab, gidx_v, sidx_v, rows_v, acc_sh, gsem, ssem, NCH)
    plsc.subcore_barrier()

    for t in range(RPT // PH):
        r0 = row0 + t * PH
        pltpu.sync_copy(acc_sh.at[pl.ds(r0, PH)], ybuf_v)
        pltpu.sync_copy(cnt.at[0, pl.ds(r0, PH)], c0_v)
        pltpu.sync_copy(cnt.at[1, pl.ds(r0, PH)], c1_v)

        def sb(r, _):
            cv = c0_v[r, pl.ds(0, 16)] + c1_v[r, pl.ds(0, 16)]
            ivv = 1.0 / jnp.maximum(cv, 1.0)
            for cc in range(D // 16):
                ybuf_v[r, pl.ds(cc * 16, 16)] = (
                    ybuf_v[r, pl.ds(cc * 16, 16)] * ivv)
            return 0

        lax.fori_loop(0, PH, sb, 0)
        for rp in range(KREP):
            pltpu.sync_copy(
                ybuf_v,
                y_out.at[pl.ds(rp * 2 * R_ACC + c * R_ACC + r0, PH)])

    for t in range(RPT // ZR):
        pltpu.sync_copy(zrow_v, acc_sh.at[pl.ds(row0 + t * ZR, ZR)])
    pltpu.sync_copy(gi2.at[c, s], gidx_v)
    pltpu.sync_copy(si2.at[s], sidx_v)
    plsc.subcore_barrier()
    _ring_loop(y_out, gidx_v, sidx_v, rows_v, acc_sh, gsem, ssem, NCH)
    plsc.subcore_barrier()
    pltpu.sync_copy(acc_sh.at[pl.ds(row0, RPT)],
                    va_out.at[c, pl.ds(row0, RPT)])


@functools.lru_cache(maxsize=None)
def _l1_fn():
    mesh = plsc.VectorSubcoreMesh(core_axis_name="c", subcore_axis_name="s",
                                  num_cores=NC, num_subcores=NS)
    out_type = (jax.ShapeDtypeStruct((KREP * 2 * R_ACC, 64), _f32),
                jax.ShapeDtypeStruct((NC, R_ACC, 64), _f32))
    scratch = [
        pltpu.VMEM((NCH, CH), _i32),
        pltpu.VMEM((NCH, CH), _i32),
        pltpu.VMEM((NB, CH, 64), _f32),
        pltpu.VMEM((ZR, 64), _f32),
        pltpu.VMEM((PH, 64), _f32),
        pltpu.VMEM((PH, 16), _f32),
        pltpu.VMEM((PH, 16), _f32),
        pltpu.VMEM_SHARED((R_ACC, 64), _f32),
        (pltpu.SemaphoreType.DMA, pltpu.SemaphoreType.DMA),
    ]
    return pl.kernel(_l1_body, out_type=out_type, mesh=mesh,
                     compiler_params=pltpu.CompilerParams(
                         use_tc_tiling_on_sc=False),
                     scratch_types=tuple(scratch))


def _mm_body(x_ref, w_ref, b_ref, o_ref):
    o_ref[...] = (jnp.dot(x_ref[...], w_ref[...],
                          preferred_element_type=_f32) + b_ref[...])


def _matmul(x, w, b, bm):
    M, K = x.shape
    N = w.shape[1]
    return pl.pallas_call(
        _mm_body,
        grid=(M // bm,),
        in_specs=[pl.BlockSpec((bm, K), lambda i: (i, 0)),
                  pl.BlockSpec((K, N), lambda i: (0, 0)),
                  pl.BlockSpec((1, N), lambda i: (0, 0))],
        out_specs=pl.BlockSpec((bm, N), lambda i: (i, 0)),
        out_shape=jax.ShapeDtypeStruct((M, N), _f32),
    )(x, w, b.reshape(1, -1))


def _merge_body(p_ref, c_ref, y_ref):
    cnt = (c_ref[0] + c_ref[1])[:, 0:1]
    y_ref[...] = p_ref[...] / jnp.maximum(cnt, 1.0)


def _merge_add_body(p_ref, q_ref, c_ref, y_ref):
    cnt = (c_ref[0] + c_ref[1])[:, 0:1]
    y_ref[...] = (p_ref[...] + q_ref[...]) / jnp.maximum(cnt, 1.0)


def _merge(p, cnt, krep, padd=None, bm=2048):
    M, D = p.shape
    nblk = R_ACC // bm
    mblk = M // bm
    body = _merge_body if padd is None else _merge_add_body
    ins = [p] if padd is None else [p, padd]
    return pl.pallas_call(
        body,
        grid=(krep * mblk,),
        in_specs=[pl.BlockSpec((bm, D), lambda i: (i % mblk, 0))] * len(ins)
                 + [pl.BlockSpec((2, bm, 16), lambda i: (0, i % nblk, 0))],
        out_specs=pl.BlockSpec((bm, D), lambda i: (i, 0)),
        out_shape=jax.ShapeDtypeStruct((krep * M, D), _f32),
    )(*ins, cnt)


def _hz_body(xa_ref, xb_ref, pa_ref, pb_ref, w2_ref, b2_ref, z_ref):
    ha = jnp.maximum(xa_ref[...] + pa_ref[...], 0.0)
    hb = jnp.maximum(xb_ref[...] + pb_ref[...], 0.0)
    w2 = w2_ref[...]
    z_ref[...] = (jnp.dot(ha, w2[:64], preferred_element_type=_f32)
                  + jnp.dot(hb, w2[64:], preferred_element_type=_f32)
                  + b2_ref[...])


def _hz(xa, xb, pa, pb, w2, b2, bm=2000):
    M, K = xa.shape
    N = w2.shape[1]
    return pl.pallas_call(
        _hz_body,
        grid=(M // bm,),
        in_specs=[pl.BlockSpec((bm, K), lambda i: (i, 0))] * 4
                 + [pl.BlockSpec((2 * K, N), lambda i: (0, 0)),
                    pl.BlockSpec((1, N), lambda i: (0, 0))],
        out_specs=pl.BlockSpec((bm, N), lambda i: (i, 0)),
        out_shape=jax.ShapeDtypeStruct((M, N), _f32),
    )(xa, xb, pa, pb, w2, b2.reshape(1, -1))


def _fin_body(z_ref, u0_ref, u1_ref, o_ref):
    o_ref[...] = z_ref[...] + u0_ref[...] + u1_ref[...]


def _fin(z, u0, u1, bm=2000):
    M, D = z.shape
    return pl.pallas_call(
        _fin_body,
        grid=(M // bm,),
        in_specs=[pl.BlockSpec((bm, D), lambda i: (i, 0))] * 3,
        out_specs=pl.BlockSpec((bm, D), lambda i: (i, 0)),
        out_shape=jax.ShapeDtypeStruct((M, D), _f32),
    )(z, u0, u1)


KREP = 8


def kernel(x0, x1, v_idx, e_idx, W1, b1, W2, b2):
    N, E = N_NODES, N_HEDGES
    ar = jnp.arange(NI_PAD - N_INC, dtype=jnp.int32)
    arf = jnp.arange(NI_PAD, dtype=jnp.int32)
    vfull = jnp.concatenate([v_idx, ar % N])
    efull = jnp.concatenate([e_idx, ar % E])
    spad = E + ar % (R_ACC - E)
    rep = arf % KREP
    gshp = (NC, NS, NCH, CH)
    sshp = (NS, NCH, CH)
    wshp = (NC * NS, NCHW, CH)
    vg = jnp.stack([vfull, vfull + N]).reshape(gshp)
    egr = efull + rep * (2 * R_ACC)
    eg = jnp.stack([egr, egr + R_ACC]).reshape(gshp)
    es_ = jnp.concatenate([e_idx, spad]).reshape(sshp)
    vs_ = jnp.concatenate([v_idx, spad]).reshape(sshp)
    vg_w = vfull.reshape(wshp)
    eg_w = (efull + rep * R_ACC).reshape(wshp)
    es_w = jnp.concatenate([e_idx, spad]).reshape(wshp)
    vs_w = jnp.concatenate([v_idx, spad]).reshape(wshp)

    s = _matmul(jnp.concatenate([x0, x1], 0), W1, b1, bm=2000)

    cnt = _counts_fn()(es_w)
    es1 = _sc_pass(s, vg, es_)
    y1 = _merge(es1.reshape(2 * R_ACC, 64), cnt, KREP)

    va1 = _sc_pass(y1, eg, vs_)
    w2p = jnp.pad(W2, ((0, 0), (0, 8)))
    b2p = jnp.pad(b2, (0, 8))
    z = _hz(s[:N], s[N:], va1[0, :N], va1[1, :N], w2p, b2p)

    es2 = _sc_pass(z, vg_w, es_w, split=True)
    y2 = _merge(es2[0], cnt, KREP, padd=es2[1])
    va2 = _sc_pass(y2, eg_w, vs_w, split=True)
    out = _fin(z, va2[0, :N], va2[1, :N])
    return out[:, :40]


# speedup vs baseline: 1.1285x; 1.1285x over previous
import functools

import jax
import jax.numpy as jnp
from jax import lax
from jax.experimental import pallas as pl
from jax.experimental.pallas import tpu as pltpu
from jax.experimental.pallas import tpu_sc as plsc

N_NODES = 10000
N_HEDGES = 10000
N_INC = 320000
NC, NS = 2, 16
CH = 128
NI_PAD = 327680
NCH = NI_PAD // NS // CH
NB = 5
NB2 = 5
R_ACC = 10240
RPT = R_ACC // NS
ZR = 128

_f32 = jnp.float32
_i32 = jnp.int32


def _sc_pass_body(split, D, *refs):
    (tab, gi, si, out,
     gidx_v, sidx_v, rows_v, zrow_v, acc_sh,
     sems) = refs
    gsem, ssem = sems
    nch = NCHW if split else NCH
    nb = NB2 if split else NB

    c = lax.axis_index("c")
    s = lax.axis_index("s")
    row0 = s * RPT

    z16 = jnp.zeros((16,), _f32)

    def zb(r, _):
        for cc in range(D // 16):
            zrow_v[r, pl.ds(cc * 16, 16)] = z16
        return 0

    lax.fori_loop(0, ZR, zb, 0)
    for t in range(RPT // ZR):
        pltpu.sync_copy(zrow_v, acc_sh.at[pl.ds(row0 + t * ZR, ZR)])

    if split:
        wid = s * NC + c
        pltpu.sync_copy(gi.at[wid], gidx_v)
        pltpu.sync_copy(si.at[wid], sidx_v)
    else:
        pltpu.sync_copy(gi.at[c, s], gidx_v)
        pltpu.sync_copy(si.at[s], sidx_v)

    plsc.subcore_barrier()

    for b in range(nb):
        pltpu.async_copy(tab.at[gidx_v.at[b]], rows_v.at[b], gsem)

    def super_step(it, _):
        j0 = it * nb
        for b in range(nb):
            j = j0 + b
            pltpu.make_async_copy(tab.at[gidx_v.at[j]], rows_v.at[b],
                                  gsem).wait()
            pltpu.async_copy(rows_v.at[b], acc_sh.at[sidx_v.at[j]], ssem,
                             add=True)
        for b in range(nb):
            j = j0 + b
            pltpu.make_async_copy(rows_v.at[b], acc_sh.at[sidx_v.at[j]],
                                  ssem).wait()
        for b in range(nb):
            jn = j0 + nb + b

            @pl.when(jn < nch)
            def _():
                pltpu.async_copy(tab.at[gidx_v.at[jn]], rows_v.at[b], gsem)
        return 0

    lax.fori_loop(0, nch // nb, super_step, 0)

    plsc.subcore_barrier()

    pltpu.sync_copy(acc_sh.at[pl.ds(row0, RPT)],
                    out.at[c, pl.ds(row0, RPT)])


@functools.lru_cache(maxsize=None)
def _sc_pass_fn(split, D):
    mesh = plsc.VectorSubcoreMesh(core_axis_name="c", subcore_axis_name="s",
                                  num_cores=NC, num_subcores=NS)
    nch = NCHW if split else NCH
    nb = NB2 if split else NB
    out_type = jax.ShapeDtypeStruct((NC, R_ACC, D), _f32)
    scratch = [
        pltpu.VMEM((nch, CH), _i32),
        pltpu.VMEM((nch, CH), _i32),
        pltpu.VMEM((nb, CH, D), _f32),
        pltpu.VMEM((ZR, D), _f32),
        pltpu.VMEM_SHARED((R_ACC, D), _f32),
        (pltpu.SemaphoreType.DMA, pltpu.SemaphoreType.DMA),
    ]
    return pl.kernel(functools.partial(_sc_pass_body, split, D),
                     out_type=out_type, mesh=mesh,
                     compiler_params=pltpu.CompilerParams(
                         use_tc_tiling_on_sc=False),
                     scratch_types=tuple(scratch))


def _sc_pass(table, gidx, sidx, split=False):
    return _sc_pass_fn(split, table.shape[1])(table, gidx, sidx)


NCHW = NI_PAD // (NC * NS) // CH


def _counts_body(*refs):
    si, cnt_out, sidx_v, zcnt_v, ones_v, cnt_sh, sems = refs
    gsem, ssem = sems
    c = lax.axis_index("c")
    s = lax.axis_index("s")
    wid = s * NC + c
    row0 = s * RPT

    z16 = jnp.zeros((16,), _f32)
    o16 = jnp.full((16,), 1.0, _f32)

    def zb(r, _):
        zcnt_v[r, pl.ds(0, 16)] = z16
        ones_v[r, pl.ds(0, 16)] = o16
        return 0

    lax.fori_loop(0, ZR, zb, 0)
    for t in range(RPT // ZR):
        pltpu.sync_copy(zcnt_v, cnt_sh.at[pl.ds(row0 + t * ZR, ZR)])
    pltpu.sync_copy(si.at[wid], sidx_v)
    plsc.subcore_barrier()

    def step(j, _):
        pltpu.sync_copy(ones_v, cnt_sh.at[sidx_v.at[j]], add=True)
        return 0

    lax.fori_loop(0, NCHW, step, 0)
    plsc.subcore_barrier()
    pltpu.sync_copy(cnt_sh.at[pl.ds(row0, RPT)],
                    cnt_out.at[c, pl.ds(row0, RPT)])


@functools.lru_cache(maxsize=None)
def _counts_fn():
    mesh = plsc.VectorSubcoreMesh(core_axis_name="c", subcore_axis_name="s",
                                  num_cores=NC, num_subcores=NS)
    scratch = [
        pltpu.VMEM((NCHW, CH), _i32),
        pltpu.VMEM((ZR, 16), _f32),
        pltpu.VMEM((CH, 16), _f32),
        pltpu.VMEM_SHARED((R_ACC, 16), _f32),
        (pltpu.SemaphoreType.DMA, pltpu.SemaphoreType.DMA),
    ]
    return pl.kernel(_counts_body,
                     out_type=jax.ShapeDtypeStruct((NC, R_ACC, 16), _f32),
                     mesh=mesh,
                     compiler_params=pltpu.CompilerParams(
                         use_tc_tiling_on_sc=False),
                     scratch_types=tuple(scratch))


def _mm_body(x_ref, w_ref, b_ref, o_ref):
    o_ref[...] = (jnp.dot(x_ref[...], w_ref[...],
                          preferred_element_type=_f32) + b_ref[...])


def _matmul(x, w, b, bm):
    M, K = x.shape
    N = w.shape[1]
    return pl.pallas_call(
        _mm_body,
        grid=(M // bm,),
        in_specs=[pl.BlockSpec((bm, K), lambda i: (i, 0)),
                  pl.BlockSpec((K, N), lambda i: (0, 0)),
                  pl.BlockSpec((1, N), lambda i: (0, 0))],
        out_specs=pl.BlockSpec((bm, N), lambda i: (i, 0)),
        out_shape=jax.ShapeDtypeStruct((M, N), _f32),
    )(x, w, b.reshape(1, -1))


def _merge_body(p_ref, c_ref, y_ref):
    cnt = (c_ref[0] + c_ref[1])[:, 0:1]
    y_ref[...] = p_ref[...] / jnp.maximum(cnt, 1.0)


def _merge_add_body(p_ref, q_ref, c_ref, y_ref):
    cnt = (c_ref[0] + c_ref[1])[:, 0:1]
    y_ref[...] = (p_ref[...] + q_ref[...]) / jnp.maximum(cnt, 1.0)


def _merge(p, cnt, krep, padd=None, bm=2048):
    M, D = p.shape
    nblk = R_ACC // bm
    mblk = M // bm
    body = _merge_body if padd is None else _merge_add_body
    ins = [p] if padd is None else [p, padd]
    return pl.pallas_call(
        body,
        grid=(krep * mblk,),
        in_specs=[pl.BlockSpec((bm, D), lambda i: (i % mblk, 0))] * len(ins)
                 + [pl.BlockSpec((2, bm, 16), lambda i: (0, i % nblk, 0))],
        out_specs=pl.BlockSpec((bm, D), lambda i: (i, 0)),
        out_shape=jax.ShapeDtypeStruct((krep * M, D), _f32),
    )(*ins, cnt)


def _hz_body(xa_ref, xb_ref, pa_ref, pb_ref, w2_ref, b2_ref, z_ref):
    ha = jnp.maximum(xa_ref[...] + pa_ref[...], 0.0)
    hb = jnp.maximum(xb_ref[...] + pb_ref[...], 0.0)
    w2 = w2_ref[...]
    z_ref[...] = (jnp.dot(ha, w2[:64], preferred_element_type=_f32)
                  + jnp.dot(hb, w2[64:], preferred_element_type=_f32)
                  + b2_ref[...])


def _hz(xa, xb, pa, pb, w2, b2, bm=2000):
    M, K = xa.shape
    N = w2.shape[1]
    return pl.pallas_call(
        _hz_body,
        grid=(M // bm,),
        in_specs=[pl.BlockSpec((bm, K), lambda i: (i, 0))] * 4
                 + [pl.BlockSpec((2 * K, N), lambda i: (0, 0)),
                    pl.BlockSpec((1, N), lambda i: (0, 0))],
        out_specs=pl.BlockSpec((bm, N), lambda i: (i, 0)),
        out_shape=jax.ShapeDtypeStruct((M, N), _f32),
    )(xa, xb, pa, pb, w2, b2.reshape(1, -1))


def _fin_body(z_ref, u0_ref, u1_ref, o_ref):
    o_ref[...] = z_ref[...] + u0_ref[...] + u1_ref[...]


def _fin(z, u0, u1, bm=2000):
    M, D = z.shape
    return pl.pallas_call(
        _fin_body,
        grid=(M // bm,),
        in_specs=[pl.BlockSpec((bm, D), lambda i: (i, 0))] * 3,
        out_specs=pl.BlockSpec((bm, D), lambda i: (i, 0)),
        out_shape=jax.ShapeDtypeStruct((M, D), _f32),
    )(z, u0, u1)


KREP = 4


def kernel(x0, x1, v_idx, e_idx, W1, b1, W2, b2):
    N, E = N_NODES, N_HEDGES
    ar = jnp.arange(NI_PAD - N_INC, dtype=jnp.int32)
    arf = jnp.arange(NI_PAD, dtype=jnp.int32)
    vfull = jnp.concatenate([v_idx, ar % N])
    efull = jnp.concatenate([e_idx, ar % E])
    spad = E + ar % (R_ACC - E)
    rep = arf % KREP
    gshp = (NC, NS, NCH, CH)
    sshp = (NS, NCH, CH)
    wshp = (NC * NS, NCHW, CH)
    vg = jnp.stack([vfull, vfull + N]).reshape(gshp)
    egr = efull + rep * (2 * R_ACC)
    eg = jnp.stack([egr, egr + R_ACC]).reshape(gshp)
    es_ = jnp.concatenate([e_idx, spad]).reshape(sshp)
    vs_ = jnp.concatenate([v_idx, spad]).reshape(sshp)
    vg_w = vfull.reshape(wshp)
    eg_w = (efull + rep * R_ACC).reshape(wshp)
    es_w = jnp.concatenate([e_idx, spad]).reshape(wshp)
    vs_w = jnp.concatenate([v_idx, spad]).reshape(wshp)

    s = _matmul(jnp.concatenate([x0, x1], 0), W1, b1, bm=2000)

    cnt = _counts_fn()(es_w)
    es1 = _sc_pass(s, vg, es_)
    y1 = _merge(es1.reshape(2 * R_ACC, 64), cnt, KREP)

    va1 = _sc_pass(y1, eg, vs_)
    w2p = jnp.pad(W2, ((0, 0), (0, 8)))
    b2p = jnp.pad(b2, (0, 8))
    z = _hz(s[:N], s[N:], va1[0, :N], va1[1, :N], w2p, b2p)

    es2 = _sc_pass(z, vg_w, es_w, split=True)
    y2 = _merge(es2[0], cnt, KREP, padd=es2[1])
    va2 = _sc_pass(y2, eg_w, vs_w, split=True)
    out = _fin(z, va2[0, :N], va2[1, :N])
    return out[:, :40]
